# Initial kernel scaffold; baseline (speedup 1.0000x reference)
import jax
import jax.numpy as jnp
from jax.experimental import pallas as pl


def kernel(x, router_w, eg, eu, ed, sg, su, sd):
    raise NotImplementedError("write your pallas kernel here")


# speedup vs baseline: 1.1573x; 1.1573x over previous
import functools

import jax
import jax.numpy as jnp
from jax import lax
from jax.experimental import pallas as pl
from jax.experimental.pallas import tpu as pltpu
from jax.experimental.pallas import tpu_sc as plsc

N = 2048
C = 768
E = 8
K = 2
H = 2048
HS = 2048
AUXC = 0.01

BLK = 128
NK = N * K
MAXB = NK // BLK + E
PMAX = MAXB * BLK
HBLK = 2048
NH = H // HBLK

SC_CORES = 2
SC_SUBCORES = 16
NW = SC_CORES * SC_SUBCORES
GCH = PMAX // NW // 2
TW = N // NW


def _router_body(xf_ref, rw_ref, idx_ref, w_ref, aux_ref):
    x = xf_ref[...]
    rw = rw_ref[...]
    logits = lax.dot_general(rw, x, (((1,), (1,)), ((), ())),
                             preferred_element_type=jnp.float32)
    e_iota = lax.broadcasted_iota(jnp.int32, (E, N), 0)
    m1 = jnp.max(logits, axis=0, keepdims=True)
    i1 = jnp.min(jnp.where(logits == m1, e_iota, E), axis=0, keepdims=True)
    masked = jnp.where(e_iota == i1, -jnp.inf, logits)
    m2 = jnp.max(masked, axis=0, keepdims=True)
    i2 = jnp.min(jnp.where(masked == m2, e_iota, E), axis=0, keepdims=True)
    t = jnp.exp(m2 - m1)
    w1 = 1.0 / (1.0 + t)
    w2 = t / (1.0 + t)
    z = jnp.exp(logits - m1)
    probs = z / jnp.sum(z, axis=0, keepdims=True)
    mean_probs = jnp.sum(probs, axis=1, keepdims=True) / N
    sel = (e_iota == i1).astype(jnp.float32) + (e_iota == i2).astype(jnp.float32)
    frac = jnp.sum(sel, axis=1, keepdims=True) / N
    aux_ref[...] = (AUXC * jnp.sum(frac * mean_probs)).reshape(1, 1)
    idx_ref[...] = jnp.concatenate([i1, i2], axis=0)
    w_ref[...] = jnp.concatenate([w1, w2], axis=0)


def _router(xf, router_w):
    return pl.pallas_call(
        _router_body,
        out_shape=(
            jax.ShapeDtypeStruct((K, N), jnp.int32),
            jax.ShapeDtypeStruct((K, N), jnp.float32),
            jax.ShapeDtypeStruct((1, 1), jnp.float32),
        ),
    )(xf, router_w)


def _shared_body(x_ref, sg_ref, su_ref, sd_ref, out_ref):
    x = x_ref[...]
    g = jnp.dot(x, sg_ref[...], preferred_element_type=jnp.float32)
    u = jnp.dot(x, su_ref[...], preferred_element_type=jnp.float32)
    h = g * jax.nn.sigmoid(g) * u
    out_ref[...] = jnp.dot(h, sd_ref[...], preferred_element_type=jnp.float32)


def _shared(xf, sg, su, sd):
    TB = 512
    return pl.pallas_call(
        _shared_body,
        grid=(N // TB,),
        in_specs=[
            pl.BlockSpec((TB, C), lambda i: (i, 0)),
            pl.BlockSpec((C, HS), lambda i: (0, 0)),
            pl.BlockSpec((C, HS), lambda i: (0, 0)),
            pl.BlockSpec((HS, C), lambda i: (0, 0)),
        ],
        out_specs=pl.BlockSpec((TB, C), lambda i: (i, 0)),
        out_shape=jax.ShapeDtypeStruct((N, C), jnp.float32),
        compiler_params=pltpu.CompilerParams(
            dimension_semantics=("arbitrary",)),
    )(xf, sg, su, sd)


def _ffn_body(be_ref, xs_ref, w_ref, eg_ref, eu_ref, ed_ref, ys_ref, acc_ref):
    i = pl.program_id(0)
    j = pl.program_id(1)

    @pl.when(be_ref[i] < E)
    def _():
        x = xs_ref[...]
        g = jnp.dot(x, eg_ref[0], preferred_element_type=jnp.float32)
        u = jnp.dot(x, eu_ref[0], preferred_element_type=jnp.float32)
        h = g * jax.nn.sigmoid(g) * u
        y = jnp.dot(h, ed_ref[0], preferred_element_type=jnp.float32)

        @pl.when(j == 0)
        def _():
            acc_ref[...] = y

        @pl.when(j > 0)
        def _():
            acc_ref[...] = acc_ref[...] + y

        @pl.when(j == NH - 1)
        def _():
            ys_ref[...] = acc_ref[...] * w_ref[0]


def _ffn(block_expert, xs, wpad, eg, eu, ed):
    def emap(i, j, be):
        return (jnp.minimum(be[i], E - 1), 0, j)

    def edmap(i, j, be):
        return (jnp.minimum(be[i], E - 1), j, 0)

    grid_spec = pltpu.PrefetchScalarGridSpec(
        num_scalar_prefetch=1,
        grid=(MAXB, NH),
        in_specs=[
            pl.BlockSpec((BLK, C), lambda i, j, be: (i, 0)),
            pl.BlockSpec((1, BLK, 1), lambda i, j, be: (i, 0, 0)),
            pl.BlockSpec((1, C, HBLK), emap),
            pl.BlockSpec((1, C, HBLK), emap),
            pl.BlockSpec((1, HBLK, C), edmap),
        ],
        out_specs=pl.BlockSpec((BLK, C), lambda i, j, be: (i, 0)),
        scratch_shapes=[pltpu.VMEM((BLK, C), jnp.float32)],
    )
    return pl.pallas_call(
        _ffn_body,
        grid_spec=grid_spec,
        out_shape=jax.ShapeDtypeStruct((PMAX, C), jnp.float32),
        compiler_params=pltpu.CompilerParams(
            dimension_semantics=("arbitrary", "arbitrary")),
    )(block_expert, xs, wpad, eg, eu, ed)


def _sc_gather(xf, gidx):
    mesh = plsc.VectorSubcoreMesh(core_axis_name="c", subcore_axis_name="s")

    @functools.partial(
        pl.kernel,
        mesh=mesh,
        out_type=jax.ShapeDtypeStruct((PMAX, C), jnp.float32),
        scratch_types=[
            pltpu.VMEM((2, GCH), jnp.int32),
            pltpu.VMEM((GCH, C), jnp.float32),
            pltpu.SemaphoreType.DMA,
        ],
    )
    def k(xf_hbm, gidx_hbm, xs_hbm, idx_v, rows_v, sem):
        wid = lax.axis_index("s") * SC_CORES + lax.axis_index("c")
        pltpu.sync_copy(gidx_hbm.at[wid], idx_v)
        for t in range(2):
            pltpu.async_copy(xf_hbm.at[idx_v.at[t]], rows_v, sem).wait()
            pltpu.sync_copy(
                rows_v, xs_hbm.at[pl.ds(wid * 2 * GCH + t * GCH, GCH)])

    return k(xf, gidx)


def _sc_combine(shared, ys, pos):
    mesh = plsc.VectorSubcoreMesh(core_axis_name="c", subcore_axis_name="s")

    @functools.partial(
        pl.kernel,
        mesh=mesh,
        out_type=jax.ShapeDtypeStruct((N, C), jnp.float32),
        scratch_types=[
            pltpu.VMEM((K, TW), jnp.int32),
            pltpu.VMEM((TW, C), jnp.float32),
            pltpu.VMEM((TW, C), jnp.float32),
            pltpu.SemaphoreType.DMA,
        ],
    )
    def k(shared_hbm, ys_hbm, pos_hbm, out_hbm, idx_v, acc_v, buf_v, sem):
        wid = lax.axis_index("s") * SC_CORES + lax.axis_index("c")
        pltpu.sync_copy(pos_hbm.at[wid], idx_v)
        pltpu.sync_copy(shared_hbm.at[pl.ds(wid * TW, TW)], acc_v)
        for t in range(K):
            pltpu.async_copy(ys_hbm.at[idx_v.at[t]], buf_v, sem).wait()

            def add_row(r, carry):
                for cc in range(C // 16):
                    sl = pl.ds(cc * 16, 16)
                    acc_v[r, sl] = acc_v[r, sl] + buf_v[r, sl]
                return carry

            lax.fori_loop(0, TW, add_row, 0)
        pltpu.sync_copy(acc_v, out_hbm.at[pl.ds(wid * TW, TW)])

    return k(shared, ys, pos)


def _dispatch_meta(idx_en, w_en):
    expert_flat = idx_en.reshape(-1)
    order = jnp.argsort(expert_flat, stable=True)
    counts = jnp.sum(expert_flat[:, None] == jnp.arange(E)[None, :],
                     axis=0).astype(jnp.int32)
    starts = jnp.concatenate([jnp.zeros((1,), jnp.int32),
                              jnp.cumsum(counts)[:-1]])
    pcounts = ((counts + BLK - 1) // BLK) * BLK
    pend = jnp.cumsum(pcounts)
    pstarts = pend - pcounts
    es = expert_flat[order]
    rank = jnp.arange(NK, dtype=jnp.int32) - starts[es]
    ppos = pstarts[es] + rank
    tok_sorted = (order % N).astype(jnp.int32)
    gidx = jnp.zeros((PMAX,), jnp.int32).at[ppos].set(tok_sorted)
    wflat = w_en.reshape(-1)[order]
    wpad = jnp.zeros((PMAX,), jnp.float32).at[ppos].set(wflat)
    pos_a = jnp.zeros((NK,), jnp.int32).at[order].set(ppos)
    block_expert = jnp.searchsorted(
        pend, jnp.arange(MAXB, dtype=jnp.int32) * BLK,
        side='right').astype(jnp.int32)
    gidx = gidx.reshape(NW, 2, GCH)
    wpad = wpad.reshape(MAXB, BLK, 1)
    pos = pos_a.reshape(K, NW, TW).transpose(1, 0, 2)
    return gidx, wpad, pos, block_expert


def kernel(x, router_w, eg, eu, ed, sg, su, sd):
    xf = x.reshape(N, C)
    idx_en, w_en, aux = _router(xf, router_w)
    gidx, wpad, pos, block_expert = _dispatch_meta(idx_en, w_en)
    xs = _sc_gather(xf, gidx)
    ys = _ffn(block_expert, xs, wpad, eg, eu, ed)
    shared = _shared(xf, sg, su, sd)
    final = _sc_combine(shared, ys, pos)
    return final.reshape(x.shape), aux[0, 0]


# speedup vs baseline: 1.2530x; 1.0827x over previous
import functools

import jax
import jax.numpy as jnp
from jax import lax
from jax.experimental import pallas as pl
from jax.experimental.pallas import tpu as pltpu
from jax.experimental.pallas import tpu_sc as plsc

N = 2048
C = 768
E = 8
K = 2
H = 2048
HS = 2048
AUXC = 0.01

BLK = 128
NK = N * K
MAXB = NK // BLK + E
PMAX = MAXB * BLK
HBLK = 2048
NH = H // HBLK

SC_CORES = 2
SC_SUBCORES = 16
NW = SC_CORES * SC_SUBCORES
GCH = PMAX // NW // 2
TW = N // NW


def _router_body(xf_ref, rw_ref, idx_ref, w_ref, aux_ref):
    x = xf_ref[...]
    rw = rw_ref[...]
    logits = lax.dot_general(rw, x, (((1,), (1,)), ((), ())),
                             preferred_element_type=jnp.float32)
    e_iota = lax.broadcasted_iota(jnp.int32, (E, N), 0)
    m1 = jnp.max(logits, axis=0, keepdims=True)
    i1 = jnp.min(jnp.where(logits == m1, e_iota, E), axis=0, keepdims=True)
    masked = jnp.where(e_iota == i1, -jnp.inf, logits)
    m2 = jnp.max(masked, axis=0, keepdims=True)
    i2 = jnp.min(jnp.where(masked == m2, e_iota, E), axis=0, keepdims=True)
    t = jnp.exp(m2 - m1)
    w1 = 1.0 / (1.0 + t)
    w2 = t / (1.0 + t)
    z = jnp.exp(logits - m1)
    probs = z / jnp.sum(z, axis=0, keepdims=True)
    mean_probs = jnp.sum(probs, axis=1, keepdims=True) / N
    sel = (e_iota == i1).astype(jnp.float32) + (e_iota == i2).astype(jnp.float32)
    frac = jnp.sum(sel, axis=1, keepdims=True) / N
    aux_ref[...] = (AUXC * jnp.sum(frac * mean_probs)).reshape(1, 1)
    idx_ref[...] = jnp.concatenate([i1, i2], axis=0)
    w_ref[...] = jnp.concatenate([w1, w2], axis=0)


def _router(xf, router_w):
    return pl.pallas_call(
        _router_body,
        out_shape=(
            jax.ShapeDtypeStruct((K, N), jnp.int32),
            jax.ShapeDtypeStruct((K, N), jnp.float32),
            jax.ShapeDtypeStruct((1, 1), jnp.float32),
        ),
    )(xf, router_w)


def _shared_body(x_ref, sg_ref, su_ref, sd_ref, out_ref):
    x = x_ref[...]
    g = jnp.dot(x, sg_ref[...], preferred_element_type=jnp.float32)
    u = jnp.dot(x, su_ref[...], preferred_element_type=jnp.float32)
    h = g * jax.nn.sigmoid(g) * u
    out_ref[...] = jnp.dot(h, sd_ref[...], preferred_element_type=jnp.float32)


def _shared(xf, sg, su, sd):
    TB = 512
    return pl.pallas_call(
        _shared_body,
        grid=(N // TB,),
        in_specs=[
            pl.BlockSpec((TB, C), lambda i: (i, 0)),
            pl.BlockSpec((C, HS), lambda i: (0, 0)),
            pl.BlockSpec((C, HS), lambda i: (0, 0)),
            pl.BlockSpec((HS, C), lambda i: (0, 0)),
        ],
        out_specs=pl.BlockSpec((TB, C), lambda i: (i, 0)),
        out_shape=jax.ShapeDtypeStruct((N, C), jnp.float32),
        compiler_params=pltpu.CompilerParams(
            dimension_semantics=("arbitrary",)),
    )(xf, sg, su, sd)


def _ffn_body(be_ref, xs_ref, w_ref, eg_ref, eu_ref, ed_ref, ys_ref, acc_ref):
    i = pl.program_id(0)
    j = pl.program_id(1)

    @pl.when(be_ref[i] < E)
    def _():
        x = xs_ref[...]
        g = jnp.dot(x, eg_ref[0], preferred_element_type=jnp.float32)
        u = jnp.dot(x, eu_ref[0], preferred_element_type=jnp.float32)
        h = g * jax.nn.sigmoid(g) * u
        y = jnp.dot(h, ed_ref[0], preferred_element_type=jnp.float32)

        @pl.when(j == 0)
        def _():
            acc_ref[...] = y

        @pl.when(j > 0)
        def _():
            acc_ref[...] = acc_ref[...] + y

        @pl.when(j == NH - 1)
        def _():
            ys_ref[...] = acc_ref[...] * w_ref[0]


def _ffn(block_expert, xs, wpad, eg, eu, ed):
    def emap(i, j, be):
        return (jnp.minimum(be[i], E - 1), 0, j)

    def edmap(i, j, be):
        return (jnp.minimum(be[i], E - 1), j, 0)

    grid_spec = pltpu.PrefetchScalarGridSpec(
        num_scalar_prefetch=1,
        grid=(MAXB, NH),
        in_specs=[
            pl.BlockSpec((BLK, C), lambda i, j, be: (i, 0)),
            pl.BlockSpec((1, BLK, 1), lambda i, j, be: (i, 0, 0)),
            pl.BlockSpec((1, C, HBLK), emap),
            pl.BlockSpec((1, C, HBLK), emap),
            pl.BlockSpec((1, HBLK, C), edmap),
        ],
        out_specs=pl.BlockSpec((BLK, C), lambda i, j, be: (i, 0)),
        scratch_shapes=[pltpu.VMEM((BLK, C), jnp.float32)],
    )
    return pl.pallas_call(
        _ffn_body,
        grid_spec=grid_spec,
        out_shape=jax.ShapeDtypeStruct((PMAX, C), jnp.float32),
        compiler_params=pltpu.CompilerParams(
            dimension_semantics=("arbitrary", "arbitrary")),
    )(block_expert, xs, wpad, eg, eu, ed)


def _sc_gather(xf, gidx):
    mesh = plsc.VectorSubcoreMesh(core_axis_name="c", subcore_axis_name="s")

    @functools.partial(
        pl.kernel,
        mesh=mesh,
        out_type=jax.ShapeDtypeStruct((PMAX, C), jnp.float32),
        scratch_types=[
            pltpu.VMEM((2, GCH), jnp.int32),
            pltpu.VMEM((GCH, C), jnp.float32),
            pltpu.VMEM((GCH, C), jnp.float32),
            pltpu.SemaphoreType.DMA,
            pltpu.SemaphoreType.DMA,
            pltpu.SemaphoreType.DMA,
            pltpu.SemaphoreType.DMA,
        ],
    )
    def k(xf_hbm, gidx_hbm, xs_hbm, idx_v, rows0, rows1, s0, s1, w0, w1):
        wid = lax.axis_index("s") * SC_CORES + lax.axis_index("c")
        pltpu.sync_copy(gidx_hbm.at[wid], idx_v)
        g0 = pltpu.async_copy(xf_hbm.at[idx_v.at[0]], rows0, s0)
        g1 = pltpu.async_copy(xf_hbm.at[idx_v.at[1]], rows1, s1)
        g0.wait()
        c0 = pltpu.async_copy(rows0, xs_hbm.at[pl.ds(wid * 2 * GCH, GCH)], w0)
        g1.wait()
        c1 = pltpu.async_copy(
            rows1, xs_hbm.at[pl.ds(wid * 2 * GCH + GCH, GCH)], w1)
        c0.wait()
        c1.wait()

    return k(xf, gidx)


def _sc_gather_out(ys, pos):
    mesh = plsc.VectorSubcoreMesh(core_axis_name="c", subcore_axis_name="s")

    @functools.partial(
        pl.kernel,
        mesh=mesh,
        out_type=jax.ShapeDtypeStruct((K * N, C), jnp.float32),
        scratch_types=[
            pltpu.VMEM((K, TW), jnp.int32),
            pltpu.VMEM((TW, C), jnp.float32),
            pltpu.VMEM((TW, C), jnp.float32),
            pltpu.SemaphoreType.DMA,
            pltpu.SemaphoreType.DMA,
            pltpu.SemaphoreType.DMA,
            pltpu.SemaphoreType.DMA,
        ],
    )
    def k(ys_hbm, pos_hbm, yg_hbm, idx_v, buf0, buf1, s0, s1, w0, w1):
        wid = lax.axis_index("s") * SC_CORES + lax.axis_index("c")
        pltpu.sync_copy(pos_hbm.at[wid], idx_v)
        g0 = pltpu.async_copy(ys_hbm.at[idx_v.at[0]], buf0, s0)
        g1 = pltpu.async_copy(ys_hbm.at[idx_v.at[1]], buf1, s1)
        g0.wait()
        c0 = pltpu.async_copy(buf0, yg_hbm.at[pl.ds(wid * TW, TW)], w0)
        g1.wait()
        c1 = pltpu.async_copy(buf1, yg_hbm.at[pl.ds(N + wid * TW, TW)], w1)
        c0.wait()
        c1.wait()

    return k(ys, pos)


def _final_body(sh_ref, y1_ref, y2_ref, out_ref):
    out_ref[...] = sh_ref[...] + y1_ref[...] + y2_ref[...]


def _final_add(shared, yg):
    TB = 512
    return pl.pallas_call(
        _final_body,
        grid=(N // TB,),
        in_specs=[
            pl.BlockSpec((TB, C), lambda i: (i, 0)),
            pl.BlockSpec((TB, C), lambda i: (i, 0)),
            pl.BlockSpec((TB, C), lambda i: (i + N // TB, 0)),
        ],
        out_specs=pl.BlockSpec((TB, C), lambda i: (i, 0)),
        out_shape=jax.ShapeDtypeStruct((N, C), jnp.float32),
        compiler_params=pltpu.CompilerParams(
            dimension_semantics=("arbitrary",)),
    )(shared, yg, yg)


def _dispatch_meta(idx_en, w_en):
    expert_flat = idx_en.reshape(-1)
    order = jnp.argsort(expert_flat, stable=True)
    counts = jnp.sum(expert_flat[:, None] == jnp.arange(E)[None, :],
                     axis=0).astype(jnp.int32)
    starts = jnp.concatenate([jnp.zeros((1,), jnp.int32),
                              jnp.cumsum(counts)[:-1]])
    pcounts = ((counts + BLK - 1) // BLK) * BLK
    pend = jnp.cumsum(pcounts)
    pstarts = pend - pcounts
    es = expert_flat[order]
    rank = jnp.arange(NK, dtype=jnp.int32) - starts[es]
    ppos = pstarts[es] + rank
    tok_sorted = (order % N).astype(jnp.int32)
    gidx = jnp.zeros((PMAX,), jnp.int32).at[ppos].set(tok_sorted)
    wflat = w_en.reshape(-1)[order]
    wpad = jnp.zeros((PMAX,), jnp.float32).at[ppos].set(wflat)
    pos_a = jnp.zeros((NK,), jnp.int32).at[order].set(ppos)
    block_expert = jnp.searchsorted(
        pend, jnp.arange(MAXB, dtype=jnp.int32) * BLK,
        side='right').astype(jnp.int32)
    gidx = gidx.reshape(NW, 2, GCH)
    wpad = wpad.reshape(MAXB, BLK, 1)
    pos = pos_a.reshape(K, NW, TW).transpose(1, 0, 2)
    return gidx, wpad, pos, block_expert


def kernel(x, router_w, eg, eu, ed, sg, su, sd):
    xf = x.reshape(N, C)
    idx_en, w_en, aux = _router(xf, router_w)
    gidx, wpad, pos, block_expert = _dispatch_meta(idx_en, w_en)
    xs = _sc_gather(xf, gidx)
    shared = _shared(xf, sg, su, sd)
    ys = _ffn(block_expert, xs, wpad, eg, eu, ed)
    yg = _sc_gather_out(ys, pos)
    final = _final_add(shared, yg)
    return final.reshape(x.shape), aux[0, 0]


# speedup vs baseline: 1.7207x; 1.3733x over previous
import functools

import jax
import jax.numpy as jnp
from jax import lax
from jax.experimental import pallas as pl
from jax.experimental.pallas import tpu as pltpu
from jax.experimental.pallas import tpu_sc as plsc

N = 2048
C = 768
E = 8
K = 2
H = 2048
HS = 2048
AUXC = 0.01

BLK = 128
NK = N * K
MAXB = NK // BLK + E
PMAX = MAXB * BLK
HBLK = 2048
NH = H // HBLK

SC_CORES = 2
SC_SUBCORES = 16
NW = SC_CORES * SC_SUBCORES
GCH = PMAX // NW // 2
TW = N // NW


def _router_body(xf_ref, rw_ref, idx_ref, w_ref, aux_ref):
    x = xf_ref[...]
    rw = rw_ref[...]
    logits = lax.dot_general(rw, x, (((1,), (1,)), ((), ())),
                             preferred_element_type=jnp.float32)
    e_iota = lax.broadcasted_iota(jnp.int32, (E, N), 0)
    m1 = jnp.max(logits, axis=0, keepdims=True)
    i1 = jnp.min(jnp.where(logits == m1, e_iota, E), axis=0, keepdims=True)
    masked = jnp.where(e_iota == i1, -jnp.inf, logits)
    m2 = jnp.max(masked, axis=0, keepdims=True)
    i2 = jnp.min(jnp.where(masked == m2, e_iota, E), axis=0, keepdims=True)
    t = jnp.exp(m2 - m1)
    w1 = 1.0 / (1.0 + t)
    w2 = t / (1.0 + t)
    z = jnp.exp(logits - m1)
    probs = z / jnp.sum(z, axis=0, keepdims=True)
    mean_probs = jnp.sum(probs, axis=1, keepdims=True) / N
    sel = (e_iota == i1).astype(jnp.float32) + (e_iota == i2).astype(jnp.float32)
    frac = jnp.sum(sel, axis=1, keepdims=True) / N
    aux_ref[...] = (AUXC * jnp.sum(frac * mean_probs)).reshape(1, 1)
    idx_ref[...] = jnp.concatenate([i1, i2], axis=0)
    w_ref[...] = jnp.concatenate([w1, w2], axis=0)


def _router(xf, router_w):
    return pl.pallas_call(
        _router_body,
        out_shape=(
            jax.ShapeDtypeStruct((K, N), jnp.int32),
            jax.ShapeDtypeStruct((K, N), jnp.float32),
            jax.ShapeDtypeStruct((1, 1), jnp.float32),
        ),
    )(xf, router_w)


def _shared_body(x_ref, sg_ref, su_ref, sd_ref, out_ref):
    x = x_ref[...]
    g = jnp.dot(x, sg_ref[...], preferred_element_type=jnp.float32)
    u = jnp.dot(x, su_ref[...], preferred_element_type=jnp.float32)
    h = g * jax.nn.sigmoid(g) * u
    out_ref[...] = jnp.dot(h, sd_ref[...], preferred_element_type=jnp.float32)


def _shared(xf, sg, su, sd):
    TB = 512
    return pl.pallas_call(
        _shared_body,
        grid=(N // TB,),
        in_specs=[
            pl.BlockSpec((TB, C), lambda i: (i, 0)),
            pl.BlockSpec((C, HS), lambda i: (0, 0)),
            pl.BlockSpec((C, HS), lambda i: (0, 0)),
            pl.BlockSpec((HS, C), lambda i: (0, 0)),
        ],
        out_specs=pl.BlockSpec((TB, C), lambda i: (i, 0)),
        out_shape=jax.ShapeDtypeStruct((N, C), jnp.float32),
        compiler_params=pltpu.CompilerParams(
            dimension_semantics=("arbitrary",)),
    )(xf, sg, su, sd)


def _ffn_body(be_ref, xs_ref, eg_ref, eu_ref, ed_ref, ys_ref, acc_ref):
    i = pl.program_id(0)
    j = pl.program_id(1)

    @pl.when(be_ref[i] < E)
    def _():
        x = xs_ref[...]
        g = jnp.dot(x, eg_ref[0], preferred_element_type=jnp.float32)
        u = jnp.dot(x, eu_ref[0], preferred_element_type=jnp.float32)
        h = g * jax.nn.sigmoid(g) * u
        y = jnp.dot(h, ed_ref[0], preferred_element_type=jnp.float32)

        @pl.when(j == 0)
        def _():
            acc_ref[...] = y

        @pl.when(j > 0)
        def _():
            acc_ref[...] = acc_ref[...] + y

        @pl.when(j == NH - 1)
        def _():
            ys_ref[...] = acc_ref[...]


def _ffn(block_expert, xs, eg, eu, ed):
    def emap(i, j, be):
        return (jnp.minimum(be[i], E - 1), 0, j)

    def edmap(i, j, be):
        return (jnp.minimum(be[i], E - 1), j, 0)

    grid_spec = pltpu.PrefetchScalarGridSpec(
        num_scalar_prefetch=1,
        grid=(MAXB, NH),
        in_specs=[
            pl.BlockSpec((BLK, C), lambda i, j, be: (i, 0)),
            pl.BlockSpec((1, C, HBLK), emap),
            pl.BlockSpec((1, C, HBLK), emap),
            pl.BlockSpec((1, HBLK, C), edmap),
        ],
        out_specs=pl.BlockSpec((BLK, C), lambda i, j, be: (i, 0)),
        scratch_shapes=[pltpu.VMEM((BLK, C), jnp.float32)],
    )
    return pl.pallas_call(
        _ffn_body,
        grid_spec=grid_spec,
        out_shape=jax.ShapeDtypeStruct((PMAX, C), jnp.float32),
        compiler_params=pltpu.CompilerParams(
            dimension_semantics=("arbitrary", "arbitrary")),
    )(block_expert, xs, eg, eu, ed)


AW = NK // NW
HC = AW // 2


def _sc_scatter_dispatch(xf, sidx):
    mesh = plsc.VectorSubcoreMesh(core_axis_name="c", subcore_axis_name="s")

    @functools.partial(
        pl.kernel,
        mesh=mesh,
        out_type=jax.ShapeDtypeStruct((PMAX, C), jnp.float32),
        scratch_types=[
            pltpu.VMEM((2, HC), jnp.int32),
            pltpu.VMEM((HC, C), jnp.float32),
            pltpu.VMEM((HC, C), jnp.float32),
            pltpu.SemaphoreType.DMA,
            pltpu.SemaphoreType.DMA,
            pltpu.SemaphoreType.DMA,
            pltpu.SemaphoreType.DMA,
        ],
    )
    def k(xf_hbm, sidx_hbm, xs_hbm, idx_v, b0, b1, s0, s1, w0, w1):
        wid = lax.axis_index("s") * SC_CORES + lax.axis_index("c")
        base = (wid * AW) % N
        pltpu.sync_copy(sidx_hbm.at[wid], idx_v)
        r0 = pltpu.async_copy(xf_hbm.at[pl.ds(base, HC)], b0, s0)
        r1 = pltpu.async_copy(xf_hbm.at[pl.ds(base + HC, HC)], b1, s1)
        r0.wait()
        c0 = pltpu.async_copy(b0, xs_hbm.at[idx_v.at[0]], w0)
        r1.wait()
        c1 = pltpu.async_copy(b1, xs_hbm.at[idx_v.at[1]], w1)
        c0.wait()
        c1.wait()

    return k(xf, sidx)


def _sc_gather_out(ys, pos):
    mesh = plsc.VectorSubcoreMesh(core_axis_name="c", subcore_axis_name="s")

    @functools.partial(
        pl.kernel,
        mesh=mesh,
        out_type=jax.ShapeDtypeStruct((K * N, C), jnp.float32),
        scratch_types=[
            pltpu.VMEM((K, TW), jnp.int32),
            pltpu.VMEM((TW, C), jnp.float32),
            pltpu.VMEM((TW, C), jnp.float32),
            pltpu.SemaphoreType.DMA,
            pltpu.SemaphoreType.DMA,
            pltpu.SemaphoreType.DMA,
            pltpu.SemaphoreType.DMA,
        ],
    )
    def k(ys_hbm, pos_hbm, yg_hbm, idx_v, buf0, buf1, s0, s1, w0, w1):
        wid = lax.axis_index("s") * SC_CORES + lax.axis_index("c")
        pltpu.sync_copy(pos_hbm.at[wid], idx_v)
        g0 = pltpu.async_copy(ys_hbm.at[idx_v.at[0]], buf0, s0)
        g1 = pltpu.async_copy(ys_hbm.at[idx_v.at[1]], buf1, s1)
        g0.wait()
        c0 = pltpu.async_copy(buf0, yg_hbm.at[pl.ds(wid * TW, TW)], w0)
        g1.wait()
        c1 = pltpu.async_copy(buf1, yg_hbm.at[pl.ds(N + wid * TW, TW)], w1)
        c0.wait()
        c1.wait()

    return k(ys, pos)


def _final_body(sh_ref, y1_ref, y2_ref, w1_ref, w2_ref, out_ref):
    out_ref[...] = (sh_ref[...] + y1_ref[...] * w1_ref[...]
                    + y2_ref[...] * w2_ref[...])


def _final_add(shared, yg, w1, w2):
    TB = 512
    return pl.pallas_call(
        _final_body,
        grid=(N // TB,),
        in_specs=[
            pl.BlockSpec((TB, C), lambda i: (i, 0)),
            pl.BlockSpec((TB, C), lambda i: (i, 0)),
            pl.BlockSpec((TB, C), lambda i: (i + N // TB, 0)),
            pl.BlockSpec((TB, 1), lambda i: (i, 0)),
            pl.BlockSpec((TB, 1), lambda i: (i, 0)),
        ],
        out_specs=pl.BlockSpec((TB, C), lambda i: (i, 0)),
        out_shape=jax.ShapeDtypeStruct((N, C), jnp.float32),
        compiler_params=pltpu.CompilerParams(
            dimension_semantics=("arbitrary",)),
    )(shared, yg, yg, w1, w2)


def _dispatch_meta(idx_en):
    expert_flat = idx_en.reshape(-1)
    onehot = (expert_flat[:, None]
              == jnp.arange(E, dtype=expert_flat.dtype)[None, :]
              ).astype(jnp.int32)
    cum = jnp.cumsum(onehot, axis=0)
    counts = cum[-1]
    pcounts = ((counts + BLK - 1) // BLK) * BLK
    pend = jnp.cumsum(pcounts)
    pstarts = pend - pcounts
    rank = jnp.take_along_axis(cum, expert_flat[:, None], axis=1)[:, 0] - 1
    ppos = pstarts[expert_flat] + rank
    block_expert = jnp.sum(
        pend[None, :] <= (jnp.arange(MAXB, dtype=jnp.int32) * BLK)[:, None],
        axis=1).astype(jnp.int32)
    sidx = ppos.reshape(NW, 2, HC)
    pos = ppos.reshape(K, NW, TW).transpose(1, 0, 2)
    return sidx, pos, block_expert


def kernel(x, router_w, eg, eu, ed, sg, su, sd):
    xf = x.reshape(N, C)
    idx_en, w_en, aux = _router(xf, router_w)
    sidx, pos, block_expert = _dispatch_meta(idx_en)
    xs = _sc_scatter_dispatch(xf, sidx)
    shared = _shared(xf, sg, su, sd)
    ys = _ffn(block_expert, xs, eg, eu, ed)
    yg = _sc_gather_out(ys, pos)
    w1 = w_en[0].reshape(N, 1)
    w2 = w_en[1].reshape(N, 1)
    final = _final_add(shared, yg, w1, w2)
    return final.reshape(x.shape), aux[0, 0]


# speedup vs baseline: 1.9174x; 1.1143x over previous
import functools

import jax
import jax.numpy as jnp
from jax import lax
from jax.experimental import pallas as pl
from jax.experimental.pallas import tpu as pltpu
from jax.experimental.pallas import tpu_sc as plsc

N = 2048
C = 768
E = 8
K = 2
H = 2048
HS = 2048
AUXC = 0.01

BLK = 256
NK = N * K
MAXB = NK // BLK + E
PMAX = MAXB * BLK
HBLK = 2048
NH = H // HBLK

SC_CORES = 2
SC_SUBCORES = 16
NW = SC_CORES * SC_SUBCORES
GCH = PMAX // NW // 2
TW = N // NW


def _router_body(xf_ref, rw_ref, idx_ref, w_ref, aux_ref):
    x = xf_ref[...]
    rw = rw_ref[...]
    logits = lax.dot_general(rw, x, (((1,), (1,)), ((), ())),
                             preferred_element_type=jnp.float32)
    e_iota = lax.broadcasted_iota(jnp.int32, (E, N), 0)
    m1 = jnp.max(logits, axis=0, keepdims=True)
    i1 = jnp.min(jnp.where(logits == m1, e_iota, E), axis=0, keepdims=True)
    masked = jnp.where(e_iota == i1, -jnp.inf, logits)
    m2 = jnp.max(masked, axis=0, keepdims=True)
    i2 = jnp.min(jnp.where(masked == m2, e_iota, E), axis=0, keepdims=True)
    t = jnp.exp(m2 - m1)
    w1 = 1.0 / (1.0 + t)
    w2 = t / (1.0 + t)
    z = jnp.exp(logits - m1)
    probs = z / jnp.sum(z, axis=0, keepdims=True)
    mean_probs = jnp.sum(probs, axis=1, keepdims=True) / N
    sel = (e_iota == i1).astype(jnp.float32) + (e_iota == i2).astype(jnp.float32)
    frac = jnp.sum(sel, axis=1, keepdims=True) / N
    aux_ref[...] = (AUXC * jnp.sum(frac * mean_probs)).reshape(1, 1)
    idx_ref[...] = jnp.concatenate([i1, i2], axis=0)
    w_ref[...] = jnp.concatenate([w1, w2], axis=0)


def _router(xf, router_w):
    return pl.pallas_call(
        _router_body,
        out_shape=(
            jax.ShapeDtypeStruct((K, N), jnp.int32),
            jax.ShapeDtypeStruct((K, N), jnp.float32),
            jax.ShapeDtypeStruct((1, 1), jnp.float32),
        ),
    )(xf, router_w)


def _shared_body(x_ref, sg_ref, su_ref, sd_ref, out_ref):
    x = x_ref[...]
    g = jnp.dot(x, sg_ref[...], preferred_element_type=jnp.float32)
    u = jnp.dot(x, su_ref[...], preferred_element_type=jnp.float32)
    h = g * jax.nn.sigmoid(g) * u
    out_ref[...] = jnp.dot(h, sd_ref[...], preferred_element_type=jnp.float32)


def _shared(xf, sg, su, sd):
    TB = 512
    return pl.pallas_call(
        _shared_body,
        grid=(N // TB,),
        in_specs=[
            pl.BlockSpec((TB, C), lambda i: (i, 0)),
            pl.BlockSpec((C, HS), lambda i: (0, 0)),
            pl.BlockSpec((C, HS), lambda i: (0, 0)),
            pl.BlockSpec((HS, C), lambda i: (0, 0)),
        ],
        out_specs=pl.BlockSpec((TB, C), lambda i: (i, 0)),
        out_shape=jax.ShapeDtypeStruct((N, C), jnp.float32),
        compiler_params=pltpu.CompilerParams(
            dimension_semantics=("arbitrary",)),
    )(xf, sg, su, sd)


def _ffn_body(be_ref, xs_ref, eg_ref, eu_ref, ed_ref, ys_ref, acc_ref):
    i = pl.program_id(0)
    j = pl.program_id(1)

    @pl.when(be_ref[i] < E)
    def _():
        x = xs_ref[...]
        g = jnp.dot(x, eg_ref[0], preferred_element_type=jnp.float32)
        u = jnp.dot(x, eu_ref[0], preferred_element_type=jnp.float32)
        h = g * jax.nn.sigmoid(g) * u
        y = jnp.dot(h, ed_ref[0], preferred_element_type=jnp.float32)

        @pl.when(j == 0)
        def _():
            acc_ref[...] = y

        @pl.when(j > 0)
        def _():
            acc_ref[...] = acc_ref[...] + y

        @pl.when(j == NH - 1)
        def _():
            ys_ref[...] = acc_ref[...]


def _ffn(block_expert, xs, eg, eu, ed):
    def emap(i, j, be):
        return (jnp.minimum(be[i], E - 1), 0, j)

    def edmap(i, j, be):
        return (jnp.minimum(be[i], E - 1), j, 0)

    grid_spec = pltpu.PrefetchScalarGridSpec(
        num_scalar_prefetch=1,
        grid=(MAXB, NH),
        in_specs=[
            pl.BlockSpec((BLK, C), lambda i, j, be: (i, 0)),
            pl.BlockSpec((1, C, HBLK), emap),
            pl.BlockSpec((1, C, HBLK), emap),
            pl.BlockSpec((1, HBLK, C), edmap),
        ],
        out_specs=pl.BlockSpec((BLK, C), lambda i, j, be: (i, 0)),
        scratch_shapes=[pltpu.VMEM((BLK, C), jnp.float32)],
    )
    return pl.pallas_call(
        _ffn_body,
        grid_spec=grid_spec,
        out_shape=jax.ShapeDtypeStruct((PMAX, C), jnp.float32),
        compiler_params=pltpu.CompilerParams(
            dimension_semantics=("arbitrary", "arbitrary")),
    )(block_expert, xs, eg, eu, ed)


AW = NK // NW
HC = AW // 2


def _sc_scatter_dispatch(xf, sidx):
    mesh = plsc.VectorSubcoreMesh(core_axis_name="c", subcore_axis_name="s")

    @functools.partial(
        pl.kernel,
        mesh=mesh,
        out_type=jax.ShapeDtypeStruct((PMAX, C), jnp.float32),
        scratch_types=[
            pltpu.VMEM((2, HC), jnp.int32),
            pltpu.VMEM((HC, C), jnp.float32),
            pltpu.VMEM((HC, C), jnp.float32),
            pltpu.SemaphoreType.DMA,
            pltpu.SemaphoreType.DMA,
            pltpu.SemaphoreType.DMA,
            pltpu.SemaphoreType.DMA,
        ],
    )
    def k(xf_hbm, sidx_hbm, xs_hbm, idx_v, b0, b1, s0, s1, w0, w1):
        wid = lax.axis_index("s") * SC_CORES + lax.axis_index("c")
        base = (wid * AW) % N
        pltpu.sync_copy(sidx_hbm.at[wid], idx_v)
        r0 = pltpu.async_copy(xf_hbm.at[pl.ds(base, HC)], b0, s0)
        r1 = pltpu.async_copy(xf_hbm.at[pl.ds(base + HC, HC)], b1, s1)
        r0.wait()
        c0 = pltpu.async_copy(b0, xs_hbm.at[idx_v.at[0]], w0)
        r1.wait()
        c1 = pltpu.async_copy(b1, xs_hbm.at[idx_v.at[1]], w1)
        c0.wait()
        c1.wait()

    return k(xf, sidx)


def _sc_gather_out(ys, pos):
    mesh = plsc.VectorSubcoreMesh(core_axis_name="c", subcore_axis_name="s")

    @functools.partial(
        pl.kernel,
        mesh=mesh,
        out_type=jax.ShapeDtypeStruct((K * N, C), jnp.float32),
        scratch_types=[
            pltpu.VMEM((K, TW), jnp.int32),
            pltpu.VMEM((TW, C), jnp.float32),
            pltpu.VMEM((TW, C), jnp.float32),
            pltpu.SemaphoreType.DMA,
            pltpu.SemaphoreType.DMA,
            pltpu.SemaphoreType.DMA,
            pltpu.SemaphoreType.DMA,
        ],
    )
    def k(ys_hbm, pos_hbm, yg_hbm, idx_v, buf0, buf1, s0, s1, w0, w1):
        wid = lax.axis_index("s") * SC_CORES + lax.axis_index("c")
        pltpu.sync_copy(pos_hbm.at[wid], idx_v)
        g0 = pltpu.async_copy(ys_hbm.at[idx_v.at[0]], buf0, s0)
        g1 = pltpu.async_copy(ys_hbm.at[idx_v.at[1]], buf1, s1)
        g0.wait()
        c0 = pltpu.async_copy(buf0, yg_hbm.at[pl.ds(wid * TW, TW)], w0)
        g1.wait()
        c1 = pltpu.async_copy(buf1, yg_hbm.at[pl.ds(N + wid * TW, TW)], w1)
        c0.wait()
        c1.wait()

    return k(ys, pos)


def _final_body(sh_ref, y1_ref, y2_ref, w1_ref, w2_ref, out_ref):
    out_ref[...] = (sh_ref[...] + y1_ref[...] * w1_ref[...]
                    + y2_ref[...] * w2_ref[...])


def _final_add(shared, yg, w1, w2):
    TB = 512
    return pl.pallas_call(
        _final_body,
        grid=(N // TB,),
        in_specs=[
            pl.BlockSpec((TB, C), lambda i: (i, 0)),
            pl.BlockSpec((TB, C), lambda i: (i, 0)),
            pl.BlockSpec((TB, C), lambda i: (i + N // TB, 0)),
            pl.BlockSpec((TB, 1), lambda i: (i, 0)),
            pl.BlockSpec((TB, 1), lambda i: (i, 0)),
        ],
        out_specs=pl.BlockSpec((TB, C), lambda i: (i, 0)),
        out_shape=jax.ShapeDtypeStruct((N, C), jnp.float32),
        compiler_params=pltpu.CompilerParams(
            dimension_semantics=("arbitrary",)),
    )(shared, yg, yg, w1, w2)


def _dispatch_meta(idx_en):
    expert_flat = idx_en.reshape(-1)
    onehot = (expert_flat[:, None]
              == jnp.arange(E, dtype=expert_flat.dtype)[None, :]
              ).astype(jnp.int32)
    cum = jnp.cumsum(onehot, axis=0)
    counts = cum[-1]
    pcounts = ((counts + BLK - 1) // BLK) * BLK
    pend = jnp.cumsum(pcounts)
    pstarts = pend - pcounts
    ppos = jnp.sum(onehot * (cum + pstarts[None, :] - 1),
                   axis=1).astype(jnp.int32)
    block_expert = jnp.sum(
        pend[None, :] <= (jnp.arange(MAXB, dtype=jnp.int32) * BLK)[:, None],
        axis=1).astype(jnp.int32)
    sidx = ppos.reshape(NW, 2, HC)
    pos = ppos.reshape(K, NW, TW).transpose(1, 0, 2)
    return sidx, pos, block_expert


def kernel(x, router_w, eg, eu, ed, sg, su, sd):
    xf = x.reshape(N, C)
    idx_en, w_en, aux = _router(xf, router_w)
    sidx, pos, block_expert = _dispatch_meta(idx_en)
    xs = _sc_scatter_dispatch(xf, sidx)
    shared = _shared(xf, sg, su, sd)
    ys = _ffn(block_expert, xs, eg, eu, ed)
    yg = _sc_gather_out(ys, pos)
    w1 = w_en[0].reshape(N, 1)
    w2 = w_en[1].reshape(N, 1)
    final = _final_add(shared, yg, w1, w2)
    return final.reshape(x.shape), aux[0, 0]
